# baseline (speedup 1.0000x reference)
import functools

import jax
import jax.numpy as jnp
from jax import lax
from jax.experimental import pallas as pl
from jax.experimental.pallas import tpu as pltpu
from jax.experimental.pallas import tpu_sc as plsc

B, P, C = 32, 20000, 21
TB = B
TPL = 3072
NB = -(-P // TPL)
LANES = 16
NC = 2
NV = P // LANES
SEARCH_ITERS = 24
NEG_ONE_BITS = -1082130432


def _tc_body(conf_ref, lab_ref, ploc_ref, gloc_ref,
             negv_ref, negb_ref, stats_ref, kv_ref):
    i = pl.program_id(0)
    x = conf_ref[...]
    lab = lab_ref[...]
    s = jnp.sum(jnp.exp(x), axis=0)
    lse = jnp.log(s)
    bg = jnp.maximum(lse - x[0], 0.0)
    cls = lax.broadcasted_iota(jnp.int32, (C, TB, TPL), 0)
    xl = jnp.sum(jnp.where(cls == lab[None], x, 0.0), axis=0)
    ce = lse - xl
    valid = (lax.broadcasted_iota(jnp.int32, (TB, TPL), 1) + i * TPL) < P
    pos = (lab > 0) & valid
    posf = jnp.where(pos, 1.0, 0.0)
    negv_ref[...] = jnp.where(pos, -1.0, bg)
    negb_ref[...] = jnp.where(pos, NEG_ONE_BITS,
                              lax.bitcast_convert_type(bg, jnp.int32))

    d = ploc_ref[...] - gloc_ref[...]
    ad = jnp.abs(d)
    sl1 = jnp.where(ad < 1.0, 0.5 * d * d, ad - 0.5)
    sl1c = sl1[:, 0] + sl1[:, 1] + sl1[:, 2] + sl1[:, 3]
    sl1_row = jnp.sum(jnp.where(pos, sl1c, 0.0), axis=1, keepdims=True)
    npos_row = jnp.sum(posf, axis=1, keepdims=True)
    pce_row = jnp.sum(jnp.where(pos, ce, 0.0), axis=1, keepdims=True)

    lane = lax.broadcasted_iota(jnp.int32, (TB, 128), 1)
    upd = (jnp.where(lane == 0, npos_row, 0.0)
           + jnp.where(lane == 1, pce_row, 0.0)
           + jnp.where(lane == 2, sl1_row, 0.0))

    @pl.when(i == 0)
    def _():
        stats_ref[...] = upd

    @pl.when(i > 0)
    def _():
        stats_ref[...] = stats_ref[...] + upd

    @pl.when(i == NB - 1)
    def _():
        st = stats_ref[...]
        np_row = st[:, 0:1]
        k_row = jnp.minimum(3.0 * np_row, jnp.float32(P) - np_row)
        kv_ref[...] = jnp.broadcast_to(k_row, (TB, 128))


def _tc_phase(conf_t, labels, ploc_t, gloc_t):
    return pl.pallas_call(
        _tc_body,
        grid=(NB,),
        in_specs=[
            pl.BlockSpec((C, TB, TPL), lambda i: (0, 0, i)),
            pl.BlockSpec((TB, TPL), lambda i: (0, i)),
            pl.BlockSpec((TB, 4, TPL), lambda i: (0, 0, i)),
            pl.BlockSpec((TB, 4, TPL), lambda i: (0, 0, i)),
        ],
        out_specs=[
            pl.BlockSpec((TB, TPL), lambda i: (0, i)),
            pl.BlockSpec((TB, TPL), lambda i: (0, i)),
            pl.BlockSpec((TB, 128), lambda i: (0, 0)),
            pl.BlockSpec((TB, 128), lambda i: (0, 0)),
        ],
        out_shape=[
            jax.ShapeDtypeStruct((B, P), jnp.float32),
            jax.ShapeDtypeStruct((B, P), jnp.int32),
            jax.ShapeDtypeStruct((B, 128), jnp.float32),
            jax.ShapeDtypeStruct((B, 128), jnp.float32),
        ],
    )(conf_t, labels, ploc_t, gloc_t)


def _sc_mesh_kernel():
    mesh = plsc.VectorSubcoreMesh(core_axis_name="c", subcore_axis_name="s")

    @functools.partial(
        pl.kernel,
        mesh=mesh,
        out_type=[
            jax.ShapeDtypeStruct((B, LANES), jnp.float32),
            jax.ShapeDtypeStruct((B, LANES), jnp.int32),
        ],
        scratch_types=[
            pltpu.VMEM((P,), jnp.float32),
            pltpu.VMEM((P,), jnp.int32),
            pltpu.VMEM((128,), jnp.float32),
            pltpu.VMEM((LANES,), jnp.float32),
            pltpu.VMEM((LANES,), jnp.int32),
            pltpu.VMEM((2 * LANES,), jnp.int32),
        ],
    )
    def topk_sum(negv_hbm, negbits_hbm, kv_hbm, sums_hbm, enc_hbm,
                 vals_v, bits_v, kv_v, sums_v, enc_v, buf_v):

        def xlane_sum_i32(x):
            for sh in (8, 4, 2, 1):
                buf_v[pl.ds(0, LANES)] = x
                buf_v[pl.ds(LANES, LANES)] = x
                x = buf_v[pl.ds(0, LANES)] + buf_v[pl.ds(sh, LANES)]
            return x

        wid = lax.axis_index("s") * NC + lax.axis_index("c")
        pltpu.sync_copy(negv_hbm.at[wid], vals_v)
        pltpu.sync_copy(negbits_hbm.at[wid], bits_v)
        pltpu.sync_copy(kv_hbm.at[wid], kv_v)
        k_vec = kv_v[pl.ds(0, LANES)].astype(jnp.int32)

        ones_i = jnp.full((LANES,), 1, jnp.int32)
        zeros_i = jnp.zeros((LANES,), jnp.int32)

        def count_ge(tv):
            def body(j, acc):
                b = bits_v[pl.ds(j * LANES, LANES)]
                return acc + jnp.where(b >= tv, ones_i, zeros_i)

            acc = lax.fori_loop(0, NV, body,
                                jnp.zeros((LANES,), jnp.int32), unroll=16)
            return xlane_sum_i32(acc)

        def bs_body(_, lohi):
            lo, hi = lohi
            mid = lo + jnp.right_shift(hi - lo + 1, 1)
            good = count_ge(mid) >= k_vec
            return (jnp.where(good, mid, lo), jnp.where(good, hi, mid - 1))

        lo0 = jnp.zeros((LANES,), jnp.int32)
        hi0 = jnp.full((LANES,), 0x7F800000, jnp.int32)
        _, hi = lax.fori_loop(0, SEARCH_ITERS, bs_body, (lo0, hi0))

        def body2(j, carry):
            sacc, cacc = carry
            b = bits_v[pl.ds(j * LANES, LANES)]
            v = vals_v[pl.ds(j * LANES, LANES)]
            gt = b > hi
            return (sacc + jnp.where(gt, v, jnp.zeros((LANES,), jnp.float32)),
                    cacc + jnp.where(gt, ones_i, zeros_i))

        sacc, cacc = lax.fori_loop(
            0, NV, body2,
            (jnp.zeros((LANES,), jnp.float32), jnp.zeros((LANES,), jnp.int32)),
            unroll=8)
        cnt_gt = xlane_sum_i32(cacc)
        lane = lax.iota(jnp.int32, LANES)
        sums_v[...] = sacc
        enc_v[...] = jnp.where(lane == 0, hi, cnt_gt)
        pltpu.sync_copy(sums_v, sums_hbm.at[wid])
        pltpu.sync_copy(enc_v, enc_hbm.at[wid])

    return topk_sum


def kernel(confidence, predicted_locations, labels, gt_locations):
    conf_t = jnp.transpose(confidence, (2, 0, 1))
    ploc_t = jnp.transpose(predicted_locations, (0, 2, 1))
    gloc_t = jnp.transpose(gt_locations, (0, 2, 1))
    negv, negbits, stats, kv = _tc_phase(conf_t, labels, ploc_t, gloc_t)
    num_pos_row = stats[:, 0]
    pce_row = stats[:, 1]
    sl1_row = stats[:, 2]
    k_row = jnp.minimum(3.0 * num_pos_row, jnp.float32(P) - num_pos_row)

    sums, enc = _sc_mesh_kernel()(negv, negbits, kv)

    t_row = lax.bitcast_convert_type(enc[:, 0], jnp.float32)
    m_row = k_row - enc[:, 1].astype(jnp.float32)
    topk_row = jnp.sum(sums, axis=1) + jnp.where(m_row > 0, m_row * t_row, 0.0)

    num_pos = jnp.sum(num_pos_row)
    cls_loss = jnp.sum(pce_row) + jnp.sum(topk_row)
    sl1_loss = jnp.sum(sl1_row)
    return sl1_loss / num_pos, cls_loss / num_pos


# speedup vs baseline: 1.0547x; 1.0547x over previous
import functools

import jax
import jax.numpy as jnp
from jax import lax
from jax.experimental import pallas as pl
from jax.experimental.pallas import tpu as pltpu
from jax.experimental.pallas import tpu_sc as plsc

B, P, C = 32, 20000, 21
TB = B
TPL = 2048
NB = -(-P // TPL)
LANES = 16
NC = 2
NV = P // LANES
SEARCH_ITERS = 20
NEG_ONE_BITS = -1082130432


def _tc_body(conf_ref, lab_ref, ploc_ref, gloc_ref,
             negv_ref, negb_ref, stats_ref, kv_ref):
    i = pl.program_id(0)
    x = conf_ref[...]
    lab = lab_ref[...]
    s = jnp.sum(jnp.exp(x), axis=0)
    lse = jnp.log(s)
    bg = jnp.maximum(lse - x[0], 0.0)
    cls = lax.broadcasted_iota(jnp.int32, (C, TB, TPL), 0)
    xl = jnp.sum(jnp.where(cls == lab[None], x, 0.0), axis=0)
    ce = lse - xl
    valid = (lax.broadcasted_iota(jnp.int32, (TB, TPL), 1) + i * TPL) < P
    pos = (lab > 0) & valid
    posf = jnp.where(pos, 1.0, 0.0)
    negv_ref[...] = jnp.where(pos, -1.0, bg)
    negb_ref[...] = jnp.where(pos, NEG_ONE_BITS,
                              lax.bitcast_convert_type(bg, jnp.int32))

    d = ploc_ref[...] - gloc_ref[...]
    ad = jnp.abs(d)
    sl1 = jnp.where(ad < 1.0, 0.5 * d * d, ad - 0.5)
    sl1c = sl1[:, 0] + sl1[:, 1] + sl1[:, 2] + sl1[:, 3]
    sl1_row = jnp.sum(jnp.where(pos, sl1c, 0.0), axis=1, keepdims=True)
    npos_row = jnp.sum(posf, axis=1, keepdims=True)
    pce_row = jnp.sum(jnp.where(pos, ce, 0.0), axis=1, keepdims=True)

    lane = lax.broadcasted_iota(jnp.int32, (TB, 128), 1)
    upd = (jnp.where(lane == 0, npos_row, 0.0)
           + jnp.where(lane == 1, pce_row, 0.0)
           + jnp.where(lane == 2, sl1_row, 0.0))

    @pl.when(i == 0)
    def _():
        stats_ref[...] = upd

    @pl.when(i > 0)
    def _():
        stats_ref[...] = stats_ref[...] + upd

    @pl.when(i == NB - 1)
    def _():
        st = stats_ref[...]
        np_row = st[:, 0:1]
        k_row = jnp.minimum(3.0 * np_row, jnp.float32(P) - np_row)
        kv_ref[...] = jnp.broadcast_to(k_row, (TB, 128))


def _tc_phase(conf_t, labels, ploc_t, gloc_t):
    return pl.pallas_call(
        _tc_body,
        grid=(NB,),
        in_specs=[
            pl.BlockSpec((C, TB, TPL), lambda i: (0, 0, i)),
            pl.BlockSpec((TB, TPL), lambda i: (0, i)),
            pl.BlockSpec((TB, 4, TPL), lambda i: (0, 0, i)),
            pl.BlockSpec((TB, 4, TPL), lambda i: (0, 0, i)),
        ],
        out_specs=[
            pl.BlockSpec((TB, TPL), lambda i: (0, i)),
            pl.BlockSpec((TB, TPL), lambda i: (0, i)),
            pl.BlockSpec((TB, 128), lambda i: (0, 0)),
            pl.BlockSpec((TB, 128), lambda i: (0, 0)),
        ],
        out_shape=[
            jax.ShapeDtypeStruct((B, P), jnp.float32),
            jax.ShapeDtypeStruct((B, P), jnp.int32),
            jax.ShapeDtypeStruct((B, 128), jnp.float32),
            jax.ShapeDtypeStruct((B, 128), jnp.float32),
        ],
    )(conf_t, labels, ploc_t, gloc_t)


def _sc_mesh_kernel():
    mesh = plsc.VectorSubcoreMesh(core_axis_name="c", subcore_axis_name="s")

    @functools.partial(
        pl.kernel,
        mesh=mesh,
        out_type=[
            jax.ShapeDtypeStruct((B, LANES), jnp.float32),
            jax.ShapeDtypeStruct((B, LANES), jnp.int32),
        ],
        scratch_types=[
            pltpu.VMEM((P,), jnp.float32),
            pltpu.VMEM((P,), jnp.int32),
            pltpu.VMEM((128,), jnp.float32),
            pltpu.VMEM((LANES,), jnp.float32),
            pltpu.VMEM((LANES,), jnp.int32),
            pltpu.VMEM((2 * LANES,), jnp.int32),
        ],
    )
    def topk_sum(negv_hbm, negbits_hbm, kv_hbm, sums_hbm, enc_hbm,
                 vals_v, bits_v, kv_v, sums_v, enc_v, buf_v):

        def xlane_sum_i32(x):
            for sh in (8, 4, 2, 1):
                buf_v[pl.ds(0, LANES)] = x
                buf_v[pl.ds(LANES, LANES)] = x
                x = buf_v[pl.ds(0, LANES)] + buf_v[pl.ds(sh, LANES)]
            return x

        wid = lax.axis_index("s") * NC + lax.axis_index("c")
        pltpu.sync_copy(negv_hbm.at[wid], vals_v)
        pltpu.sync_copy(negbits_hbm.at[wid], bits_v)
        pltpu.sync_copy(kv_hbm.at[wid], kv_v)
        k_vec = kv_v[pl.ds(0, LANES)].astype(jnp.int32)

        ones_i = jnp.full((LANES,), 1, jnp.int32)
        zeros_i = jnp.zeros((LANES,), jnp.int32)

        def count_ge(tv):
            def body(j, acc):
                b = bits_v[pl.ds(j * LANES, LANES)]
                return acc + jnp.where(b >= tv, ones_i, zeros_i)

            acc = lax.fori_loop(0, NV, body,
                                jnp.zeros((LANES,), jnp.int32), unroll=8)
            return xlane_sum_i32(acc)

        def bs_body(_, lohi):
            lo, hi = lohi
            mid = lo + jnp.right_shift(hi - lo + 1, 1)
            good = count_ge(mid) >= k_vec
            return (jnp.where(good, mid, lo), jnp.where(good, hi, mid - 1))

        lo0 = jnp.zeros((LANES,), jnp.int32)
        hi0 = jnp.full((LANES,), 0x7F800000, jnp.int32)
        _, hi = lax.fori_loop(0, SEARCH_ITERS, bs_body, (lo0, hi0))

        def body2(j, carry):
            sacc, cacc = carry
            b = bits_v[pl.ds(j * LANES, LANES)]
            v = vals_v[pl.ds(j * LANES, LANES)]
            gt = b > hi
            return (sacc + jnp.where(gt, v, jnp.zeros((LANES,), jnp.float32)),
                    cacc + jnp.where(gt, ones_i, zeros_i))

        sacc, cacc = lax.fori_loop(
            0, NV, body2,
            (jnp.zeros((LANES,), jnp.float32), jnp.zeros((LANES,), jnp.int32)),
            unroll=8)
        cnt_gt = xlane_sum_i32(cacc)
        lane = lax.iota(jnp.int32, LANES)
        sums_v[...] = sacc
        enc_v[...] = jnp.where(lane == 0, hi, cnt_gt)
        pltpu.sync_copy(sums_v, sums_hbm.at[wid])
        pltpu.sync_copy(enc_v, enc_hbm.at[wid])

    return topk_sum


def kernel(confidence, predicted_locations, labels, gt_locations):
    conf_t = jnp.transpose(confidence, (2, 0, 1))
    ploc_t = jnp.transpose(predicted_locations, (0, 2, 1))
    gloc_t = jnp.transpose(gt_locations, (0, 2, 1))
    negv, negbits, stats, kv = _tc_phase(conf_t, labels, ploc_t, gloc_t)
    num_pos_row = stats[:, 0]
    pce_row = stats[:, 1]
    sl1_row = stats[:, 2]
    k_row = jnp.minimum(3.0 * num_pos_row, jnp.float32(P) - num_pos_row)

    sums, enc = _sc_mesh_kernel()(negv, negbits, kv)

    t_row = lax.bitcast_convert_type(enc[:, 0], jnp.float32)
    m_row = k_row - enc[:, 1].astype(jnp.float32)
    topk_row = jnp.sum(sums, axis=1) + jnp.where(m_row > 0, m_row * t_row, 0.0)

    num_pos = jnp.sum(num_pos_row)
    cls_loss = jnp.sum(pce_row) + jnp.sum(topk_row)
    sl1_loss = jnp.sum(sl1_row)
    return sl1_loss / num_pos, cls_loss / num_pos
